# baseline (speedup 1.0000x reference)
import functools

import jax
import jax.numpy as jnp
from jax import lax
from jax.experimental import pallas as pl
from jax.experimental.pallas import tpu as pltpu
from jax.experimental.pallas import tpu_sc as plsc

_NW = 32
_SUB = 1024

_SC_PARAMS = pltpu.CompilerParams(
    use_tc_tiling_on_sc=False, needs_layout_passes=False
)


_CORNERS = ((0, 0), (0, 1), (1, 0), (1, 1))


def _index_body(W, H, sgx_ref, sgy_ref, flow_ref, out_ref):
    fx = flow_ref[0, 0]
    fy = flow_ref[0, 1]
    gx = sgx_ref[0] + fx.T
    gy = sgy_ref[0] + fy.T
    ix = jnp.clip(jnp.round((gx + 1.0) * 0.5 * (W - 1)), 0, W - 1).astype(jnp.int32)
    iy = jnp.clip(jnp.round((gy + 1.0) * 0.5 * (H - 1)), 0, H - 1).astype(jnp.int32)
    idx = iy * W + ix
    spread = lax.broadcasted_iota(jnp.int32, idx.shape, 1) & 63
    for k, (cy, cx) in enumerate(_CORNERS):
        cidx = cy * (H - 1) * W + cx * (W - 1)
        idx = jnp.where(idx == cidx, H * W + k * 64 + spread, idx)
    out_ref[0] = idx


def _make_index_kernel(B, H, W, hb):
    return pl.pallas_call(
        functools.partial(_index_body, W, H),
        grid=(B, H // hb),
        in_specs=[
            pl.BlockSpec((1, hb, W), lambda b, i: (b, i, 0)),
            pl.BlockSpec((1, hb, W), lambda b, i: (b, i, 0)),
            pl.BlockSpec((1, 2, W, hb), lambda b, i: (b, 0, 0, i)),
        ],
        out_specs=pl.BlockSpec((1, hb, W), lambda b, i: (b, i, 0)),
        out_shape=jax.ShapeDtypeStruct((B, H, W), jnp.int32),
    )


def _wid():
    return lax.axis_index("s") * 2 + lax.axis_index("c")


def _make_pass1(B, C, H, W):
    HW = H * W
    chunk = HW // _NW
    nsub = chunk // _SUB
    mesh = plsc.VectorSubcoreMesh(core_axis_name="c", subcore_axis_name="s")

    @functools.partial(
        pl.kernel,
        mesh=mesh,
        compiler_params=_SC_PARAMS,
        out_type=jax.ShapeDtypeStruct((B, HW + 256, C), jnp.float32),
        scratch_types=[
            pltpu.VMEM((3, C, _SUB + 8), jnp.float32),
            pltpu.VMEM((3, _SUB, C), jnp.float32),
            pltpu.VMEM((128, C), jnp.float32),
            pltpu.SemaphoreType.DMA,
            pltpu.SemaphoreType.DMA,
            pltpu.SemaphoreType.DMA,
            pltpu.SemaphoreType.DMA,
            pltpu.SemaphoreType.DMA,
            pltpu.SemaphoreType.DMA,
        ],
    )
    def pass1(x_hbm, tab_hbm, in_v, rows_v, rep_v, is0, is1, is2, os0, os1, os2):
        base = _wid() * chunk
        isems = (is0, is1, is2)
        osems = (os0, os1, os2)
        iota = lax.iota(jnp.int32, 16)

        wid = _wid()

        def write_corner_replicas(b, sl, local_a, local_b, spare_off):
            va = rows_v[sl, local_a, :]
            vb = rows_v[sl, local_b, :]

            def rep_body(r, _):
                rep_v[r, :] = va
                rep_v[64 + r, :] = vb
                return 0

            lax.fori_loop(0, 64, rep_body, 0)
            pltpu.sync_copy(rep_v, tab_hbm.at[b, pl.ds(HW + spare_off, 128), :])

        def start_in(b, s, sl):
            return pltpu.async_copy(
                x_hbm.at[b, :, pl.ds(base + s * _SUB, _SUB)],
                in_v.at[sl, :, pl.ds(0, _SUB)],
                isems[sl],
            )

        pend_in = {}
        pend_out = {}

        def ensure_free(sl):
            if sl in pend_out:
                pend_out.pop(sl).wait()

        for b in range(B):
            for s in range(nsub):
                sl = s % 3
                if s == 0:
                    for t in range(min(3, nsub)):
                        pend_in[t % 3] = start_in(b, t, t % 3)
                elif s + 2 < nsub:
                    pend_in[(s + 2) % 3] = start_in(b, s + 2, (s + 2) % 3)
                pend_in.pop(sl).wait()
                ensure_free(sl)

                @plsc.parallel_loop(0, _SUB, unroll=8)
                def _(p):
                    v = plsc.load_gather(
                        in_v.at[sl], [iota, jnp.broadcast_to(p, (16,))]
                    )
                    rows_v[sl, p, :] = v

                pend_out[sl] = pltpu.async_copy(
                    rows_v.at[sl],
                    tab_hbm.at[b, pl.ds(base + s * _SUB, _SUB), :],
                    osems[sl],
                )
                if s == 0:
                    @pl.when(wid == 0)
                    def _():
                        write_corner_replicas(b, sl, 0, W - 1, 0)

                if s == nsub - 1:
                    @pl.when(wid == _NW - 1)
                    def _():
                        write_corner_replicas(
                            b,
                            sl,
                            (H - 1) * W - (_NW - 1) * chunk - (nsub - 1) * _SUB,
                            chunk - (nsub - 1) * _SUB - 1,
                            128,
                        )
        for t in range(3):
            ensure_free(t)

    return pass1


def _make_pass2(B, C, HW, ntab):
    chunk = HW // _NW
    nsub = chunk // _SUB
    mesh = plsc.VectorSubcoreMesh(core_axis_name="c", subcore_axis_name="s")

    @functools.partial(
        pl.kernel,
        mesh=mesh,
        compiler_params=_SC_PARAMS,
        out_type=jax.ShapeDtypeStruct((B, C, HW), jnp.float32),
        scratch_types=[
            pltpu.VMEM((chunk,), jnp.int32),
            pltpu.VMEM((3, _SUB, C), jnp.float32),
            pltpu.VMEM((3, C, _SUB), jnp.float32),
            pltpu.SemaphoreType.DMA,
            pltpu.SemaphoreType.DMA,
            pltpu.SemaphoreType.DMA,
            pltpu.SemaphoreType.DMA,
            pltpu.SemaphoreType.DMA,
            pltpu.SemaphoreType.DMA,
        ],
    )
    def pass2(tab_hbm, idx_hbm, out_hbm, idx_v, rows_v, pla_v, is0, is1, is2, os0, os1, os2):
        base = _wid() * chunk
        isems = (is0, is1, is2)
        osems = (os0, os1, os2)
        iota = lax.iota(jnp.int32, 16)

        def start_gather(b, s, sl):
            return pltpu.async_copy(
                tab_hbm.at[b].at[idx_v.at[pl.ds(s * _SUB, _SUB)]],
                rows_v.at[sl],
                isems[sl],
            )

        pend_in = {}
        pend_out = {}

        def ensure_free(sl):
            if sl in pend_out:
                pend_out.pop(sl).wait()

        for b in range(B):
            pltpu.sync_copy(idx_hbm.at[b, pl.ds(base, chunk)], idx_v)
            for s in range(nsub):
                sl = s % 3
                if s == 0:
                    for t in range(min(3, nsub)):
                        pend_in[t % 3] = start_gather(b, t, t % 3)
                elif s + 2 < nsub:
                    pend_in[(s + 2) % 3] = start_gather(b, s + 2, (s + 2) % 3)
                pend_in.pop(sl).wait()
                ensure_free(sl)

                @plsc.parallel_loop(0, _SUB, unroll=8)
                def _(j):
                    c = j & 15
                    p0 = j - c
                    v = plsc.load_gather(
                        rows_v.at[sl], [p0 + iota, jnp.broadcast_to(c, (16,))]
                    )
                    pla_v[sl, c, pl.ds(p0, 16)] = v

                pend_out[sl] = pltpu.async_copy(
                    pla_v.at[sl],
                    out_hbm.at[b, :, pl.ds(base + s * _SUB, _SUB)],
                    osems[sl],
                )
        for t in range(3):
            ensure_free(t)

    return pass2


def kernel(x, flow, sample_grid):
    B, C, H, W = x.shape
    HW = H * W
    sgx = sample_grid[..., 0]
    sgy = sample_grid[..., 1]
    idx = _make_index_kernel(B, H, W, 128)(sgx, sgy, flow)
    table = _make_pass1(B, C, H, W)(x.reshape(B, C, HW))
    out = _make_pass2(B, C, HW, HW + 256)(table, idx.reshape(B, HW))
    return out.reshape(B, C, H, W)
